# baseline (speedup 1.0000x reference)
import functools

import jax
import jax.numpy as jnp
from jax import lax
from jax.experimental import pallas as pl
from jax.experimental.pallas import tpu as pltpu
from jax.experimental.pallas import tpu_sc as plsc


def _pack_bf16_pairs(v):
    d = v.shape[-1]
    lo = lax.bitcast_convert_type(
        v[:, :d // 2].astype(jnp.bfloat16), jnp.uint16).astype(jnp.uint32)
    hi = lax.bitcast_convert_type(
        v[:, d // 2:].astype(jnp.bfloat16), jnp.uint16).astype(jnp.uint32)
    return ((hi << 16) | lo).astype(jnp.int32)


def _edge_lin_body(at_ref, w_ref, b_ref, o_ref):
    ev = (
        lax.dot_general(at_ref[...], w_ref[...],
                        (((0,), (0,)), ((), ())),
                        preferred_element_type=jnp.float32)
        + b_ref[...]
    )
    evp = ev.reshape(ev.shape[0] // 2, 2, ev.shape[1])
    o_ref[...] = jnp.concatenate(
        [_pack_bf16_pairs(evp[:, 0, :]), _pack_bf16_pairs(evp[:, 1, :])],
        axis=1)


def _edge_linear(edge_attr_t, W_edge, b_edge, ep, block_e):
    DE, E = edge_attr_t.shape
    D = W_edge.shape[1]
    grid = ep // block_e
    return pl.pallas_call(
        _edge_lin_body,
        grid=(grid,),
        in_specs=[
            pl.BlockSpec((DE, block_e), lambda i: (0, i)),
            pl.BlockSpec((DE, D), lambda i: (0, 0)),
            pl.BlockSpec((1, D), lambda i: (0, 0)),
        ],
        out_specs=pl.BlockSpec((block_e // 2, D), lambda i: (i, 0)),
        out_shape=jax.ShapeDtypeStruct((ep // 2, D), jnp.int32),
    )(edge_attr_t, W_edge, b_edge.reshape(1, D))


def _sc_aggregate(x, src1, dst1, e, zeros, nch, cb):
    N = x.shape[0]
    NPAD, D = zeros.shape
    NCH, CB = nch, cb
    info = plsc.get_sparse_core_info()
    NC, NS = info.num_cores, info.num_subcores
    NW = NC * NS
    EPT = NCH * CB
    assert src1.shape[0] == NW * EPT and NCH % 2 == 0 and CB % 8 == 0
    assert NPAD % (8 * NS) == 0
    RPT = NPAD // NS

    mesh = plsc.VectorSubcoreMesh(core_axis_name="c", subcore_axis_name="s")

    @functools.partial(
        pl.kernel,
        out_type=jax.ShapeDtypeStruct((NC, NPAD, D), jnp.float32),
        mesh=mesh,
        scratch_types=[
            pltpu.VMEM((4, CB), jnp.int32),
            pltpu.VMEM((4, CB), jnp.int32),
            pltpu.VMEM((CB, D), jnp.float32),
            pltpu.VMEM((CB, D), jnp.float32),
            pltpu.VMEM((CB // 2, D), jnp.int32),
            pltpu.VMEM((CB // 2, D), jnp.int32),
            pltpu.VMEM_SHARED((NPAD, D), jnp.float32),
            pltpu.SemaphoreType.DMA,
            pltpu.SemaphoreType.DMA,
            pltpu.SemaphoreType.DMA,
            pltpu.SemaphoreType.DMA,
            pltpu.SemaphoreType.DMA,
            pltpu.SemaphoreType.DMA,
            pltpu.SemaphoreType.DMA,
            pltpu.SemaphoreType.DMA,
            pltpu.SemaphoreType.DMA,
            pltpu.SemaphoreType.DMA,
        ],
    )
    def body(x_hbm, src_hbm, dst_hbm, e_hbm, zero_hbm, out_hbm,
             srcb, dstb, xb0, xb1, eb0, eb1, acc,
             gsem0, gsem1, esem0, esem1, ssem0, ssem1,
             isem0, isem1, isem2, isem3):
        c = lax.axis_index("c")
        s = lax.axis_index("s")
        wid = c * NS + s
        xb = (xb0, xb1)
        eb = (eb0, eb1)
        gsem = (gsem0, gsem1)
        esem = (esem0, esem1)
        ssem = (ssem0, ssem1)
        isem = (isem0, isem1, isem2, isem3)

        pltpu.sync_copy(zero_hbm.at[pl.ds(s * RPT, RPT)],
                        acc.at[pl.ds(s * RPT, RPT)])
        plsc.subcore_barrier()

        ebase = wid * EPT

        def i_slice(hbm, i):
            return hbm.at[pl.ds(pl.multiple_of(ebase + i * CB, 8), CB)]

        def e_slice(i):
            return e_hbm.at[
                pl.ds(pl.multiple_of((ebase + i * CB) // 2, 8), CB // 2)]

        def idx_issue(i, q):
            pltpu.async_copy(i_slice(src_hbm, i), srcb.at[q], isem[q])
            pltpu.async_copy(i_slice(dst_hbm, i), dstb.at[q], isem[q])

        def idx_wait(i, q):
            pltpu.make_async_copy(i_slice(src_hbm, i), srcb.at[q],
                                  isem[q]).wait()
            pltpu.make_async_copy(i_slice(dst_hbm, i), dstb.at[q],
                                  isem[q]).wait()

        def gather_issue(i, q, p):
            pltpu.async_copy(x_hbm.at[srcb.at[q]], xb[p], gsem[p])

        def compute(p):
            xbp = xb[p]
            ebp = eb[p]

            mask = jnp.full((16,), -0x10000, jnp.int32)

            def as_f32(bits):
                return lax.bitcast_convert_type(bits, jnp.float32)

            def rowpair(k, carry):
                for je in range(2):
                    r = 2 * k + je
                    for j in range(D // 32):
                        lo_sl = pl.ds(j * 16, 16)
                        hi_sl = pl.ds(D // 2 + j * 16, 16)
                        ew = ebp[k, pl.ds(je * (D // 2) + j * 16, 16)]
                        lo = xbp[r, lo_sl] + as_f32(ew << 16)
                        hi = xbp[r, hi_sl] + as_f32(ew & mask)
                        xbp[r, lo_sl] = jnp.maximum(lo, 0.0)
                        xbp[r, hi_sl] = jnp.maximum(hi, 0.0)
                return carry

            lax.fori_loop(0, CB // 2, rowpair, 0)

        def step(i, p, q, pf2, pf4):
            pltpu.make_async_copy(x_hbm.at[srcb.at[q]], xb[p], gsem[p]).wait()
            pltpu.make_async_copy(e_slice(i), eb[p], esem[p]).wait()
            compute(p)
            pltpu.async_copy(xb[p], acc.at[dstb.at[q]], ssem[p], add=True)
            if pf2:
                q2 = (q + 2) % 4
                idx_wait(i + 2, q2)
                pltpu.async_copy(e_slice(i + 2), eb[p], esem[p])
            pltpu.make_async_copy(xb[p], acc.at[dstb.at[q]], ssem[p]).wait()
            if pf2:
                gather_issue(i + 2, (q + 2) % 4, p)
            if pf4:
                idx_issue(i + 4, q)

        for q in range(4):
            idx_issue(q, q)
        idx_wait(0, 0)
        gather_issue(0, 0, 0)
        pltpu.async_copy(e_slice(0), eb[0], esem[0])
        idx_wait(1, 1)
        gather_issue(1, 1, 1)
        pltpu.async_copy(e_slice(1), eb[1], esem[1])

        assert NCH % 4 == 0

        def quad(j, carry):
            i0 = j * 4
            for k in range(4):
                step(i0 + k, k % 2, k, True, True)
            return carry

        lax.fori_loop(0, NCH // 4 - 1, quad, 0)
        i0 = NCH - 4
        step(i0, 0, 0, True, False)
        step(i0 + 1, 1, 1, True, False)
        step(i0 + 2, 0, 2, False, False)
        step(i0 + 3, 1, 3, False, False)

        plsc.subcore_barrier()
        pltpu.sync_copy(acc.at[pl.ds(s * RPT, RPT)],
                        out_hbm.at[c, pl.ds(s * RPT, RPT)])

    return body(x, src1, dst1, e, zeros)


def _final_body(x_ref, p_ref, w_ref, b_ref, o_ref):
    h = x_ref[...] + p_ref[0] + p_ref[1]
    o_ref[...] = (
        jnp.dot(h, w_ref[...], preferred_element_type=jnp.float32)
        + b_ref[...]
    )


def _final(x, parts, W, b, block_n):
    N, D = x.shape
    grid = N // block_n
    return pl.pallas_call(
        _final_body,
        grid=(grid,),
        in_specs=[
            pl.BlockSpec((block_n, D), lambda i: (i, 0)),
            pl.BlockSpec((2, block_n, D), lambda i: (0, i, 0)),
            pl.BlockSpec((D, D), lambda i: (0, 0)),
            pl.BlockSpec((1, D), lambda i: (0, 0)),
        ],
        out_specs=pl.BlockSpec((block_n, D), lambda i: (i, 0)),
        out_shape=jax.ShapeDtypeStruct((N, D), jnp.float32),
    )(x, parts, W, b.reshape(1, D))


def kernel(x, edge_index, edge_attr, y, W_edge, b_edge, W, b):
    N, D = x.shape
    E = edge_index.shape[1]
    NW, NCH, CB = 32, 128, 80
    ep = NW * NCH * CB
    npad = -(-N // 128) * 128
    src = edge_index[0]
    dst = edge_index[1]
    pad = ep - E
    pidx = jnp.arange(pad, dtype=jnp.int32)
    src1 = jnp.concatenate([src, pidx % N])
    dst1 = jnp.concatenate([dst, N + pidx % (npad - N)])
    e = _edge_linear(edge_attr.T, W_edge, b_edge, ep, block_e=2560)
    zeros = jnp.zeros((npad, D), jnp.float32)
    parts = _sc_aggregate(x, src1, dst1, e, zeros, NCH, CB)
    pred = _final(x, parts, W, b, block_n=1000)
    return (pred, y)


# speedup vs baseline: 2.1691x; 2.1691x over previous
import functools

import jax
import jax.numpy as jnp
from jax import lax
from jax.experimental import pallas as pl
from jax.experimental.pallas import tpu as pltpu
from jax.experimental.pallas import tpu_sc as plsc


def _pack_bf16_pairs(lo_f32, hi_f32):
    lo = lax.bitcast_convert_type(
        lo_f32.astype(jnp.bfloat16), jnp.uint16).astype(jnp.uint32)
    hi = lax.bitcast_convert_type(
        hi_f32.astype(jnp.bfloat16), jnp.uint16).astype(jnp.uint32)
    return ((hi << 16) | lo).astype(jnp.int32)


def _make_edge_lin_body(cb):
    def body(at_ref, wlo_ref, whi_ref, blo_ref, bhi_ref, o_ref):
        dn = (((0,), (0,)), ((), ()))
        e_lo = lax.dot_general(at_ref[...], wlo_ref[...], dn,
                               preferred_element_type=jnp.float32) + blo_ref[...]
        e_hi = lax.dot_general(at_ref[...], whi_ref[...], dn,
                               preferred_element_type=jnp.float32) + bhi_ref[...]
        packed = _pack_bf16_pairs(e_lo, e_hi)
        be, dh = packed.shape
        pr = packed.reshape(be // cb, 2, cb // 2, dh)
        o_ref[...] = jnp.concatenate([pr[:, 0], pr[:, 1]],
                                     axis=-1).reshape(be // 2, 2 * dh)
    return body


def _edge_linear(edge_attr_t, W_edge, b_edge, ep, block_e, cb):
    DE, E = edge_attr_t.shape
    D = W_edge.shape[1]
    grid = ep // block_e
    return pl.pallas_call(
        _make_edge_lin_body(cb),
        grid=(grid,),
        in_specs=[
            pl.BlockSpec((DE, block_e), lambda i: (0, i)),
            pl.BlockSpec((DE, D // 2), lambda i: (0, 0)),
            pl.BlockSpec((DE, D // 2), lambda i: (0, 0)),
            pl.BlockSpec((1, D // 2), lambda i: (0, 0)),
            pl.BlockSpec((1, D // 2), lambda i: (0, 0)),
        ],
        out_specs=pl.BlockSpec((block_e // 2, D), lambda i: (i, 0)),
        out_shape=jax.ShapeDtypeStruct((ep // 2, D), jnp.int32),
    )(edge_attr_t, W_edge[:, :D // 2], W_edge[:, D // 2:],
      b_edge[:D // 2].reshape(1, D // 2), b_edge[D // 2:].reshape(1, D // 2))


def _sc_aggregate(x, src1, dst1, e, zeros, nch, cb):
    N = x.shape[0]
    NPAD, D = zeros.shape
    NCH, CB = nch, cb
    info = plsc.get_sparse_core_info()
    NC, NS = info.num_cores, info.num_subcores
    NW = NC * NS
    EPT = NCH * CB
    assert src1.shape[0] == NW * EPT and NCH % 2 == 0 and CB % 8 == 0
    assert NPAD % (8 * NS) == 0
    RPT = NPAD // NS

    mesh = plsc.VectorSubcoreMesh(core_axis_name="c", subcore_axis_name="s")

    @functools.partial(
        pl.kernel,
        out_type=jax.ShapeDtypeStruct((NC, NPAD, D), jnp.float32),
        mesh=mesh,
        scratch_types=[
            pltpu.VMEM((4, CB), jnp.int32),
            pltpu.VMEM((4, CB), jnp.int32),
            pltpu.VMEM((CB, D), jnp.float32),
            pltpu.VMEM((CB, D), jnp.float32),
            pltpu.VMEM((CB // 2, D), jnp.int32),
            pltpu.VMEM((CB // 2, D), jnp.int32),
            pltpu.VMEM((CB, D), jnp.float32),
            pltpu.VMEM((CB, D), jnp.float32),
            pltpu.VMEM_SHARED((NPAD, D), jnp.float32),
            pltpu.SemaphoreType.DMA,
            pltpu.SemaphoreType.DMA,
            pltpu.SemaphoreType.DMA,
            pltpu.SemaphoreType.DMA,
            pltpu.SemaphoreType.DMA,
            pltpu.SemaphoreType.DMA,
            pltpu.SemaphoreType.DMA,
            pltpu.SemaphoreType.DMA,
            pltpu.SemaphoreType.DMA,
            pltpu.SemaphoreType.DMA,
        ],
    )
    def body(x_hbm, src_hbm, dst_hbm, e_hbm, zero_hbm, out_hbm,
             srcb, dstb, xb0, xb1, eb0, eb1, mb0, mb1, acc,
             gsem0, gsem1, esem0, esem1, ssem0, ssem1,
             isem0, isem1, isem2, isem3):
        c = lax.axis_index("c")
        s = lax.axis_index("s")
        wid = c * NS + s
        xb = (xb0, xb1)
        eb = (eb0, eb1)
        mb = (mb0, mb1)
        gsem = (gsem0, gsem1)
        esem = (esem0, esem1)
        ssem = (ssem0, ssem1)
        isem = (isem0, isem1, isem2, isem3)

        pltpu.sync_copy(zero_hbm.at[pl.ds(s * RPT, RPT)],
                        acc.at[pl.ds(s * RPT, RPT)])
        plsc.subcore_barrier()

        ebase = wid * EPT

        def i_slice(hbm, i):
            return hbm.at[pl.ds(pl.multiple_of(ebase + i * CB, 8), CB)]

        def e_slice(i):
            return e_hbm.at[
                pl.ds(pl.multiple_of((ebase + i * CB) // 2, 8), CB // 2)]

        def idx_issue(i, q):
            pltpu.async_copy(i_slice(src_hbm, i), srcb.at[q], isem[q])
            pltpu.async_copy(i_slice(dst_hbm, i), dstb.at[q], isem[q])

        def idx_wait(i, q):
            pltpu.make_async_copy(i_slice(src_hbm, i), srcb.at[q],
                                  isem[q]).wait()
            pltpu.make_async_copy(i_slice(dst_hbm, i), dstb.at[q],
                                  isem[q]).wait()

        def gather_issue(i, q, p):
            pltpu.async_copy(x_hbm.at[srcb.at[q]], xb[p], gsem[p])

        def compute(p):
            xbp = xb[p]
            ebp = eb[p]
            mbp = mb[p]

            mask = jnp.full((16,), -0x10000, jnp.int32)

            def as_f32(bits):
                return lax.bitcast_convert_type(bits, jnp.float32)

            def rowpair(k, carry):
                for je in range(2):
                    r = k + je * (CB // 2)
                    for j in range(D // 32):
                        lo_sl = pl.ds(j * 16, 16)
                        hi_sl = pl.ds(D // 2 + j * 16, 16)
                        ew = ebp[k, pl.ds(je * (D // 2) + j * 16, 16)]
                        lo = xbp[r, lo_sl] + as_f32(ew << 16)
                        hi = xbp[r, hi_sl] + as_f32(ew & mask)
                        mbp[r, lo_sl] = jnp.maximum(lo, 0.0)
                        mbp[r, hi_sl] = jnp.maximum(hi, 0.0)
                return carry

            lax.fori_loop(0, CB // 2, rowpair, 0)

        def step(i, p, q, pf2, pf4):
            pltpu.make_async_copy(x_hbm.at[srcb.at[q]], xb[p], gsem[p]).wait()
            pltpu.make_async_copy(e_slice(i), eb[p], esem[p]).wait()
            compute(p)
            pltpu.async_copy(mb[p], acc.at[dstb.at[q]], ssem[p], add=True)
            if pf2:
                q2 = (q + 2) % 4
                idx_wait(i + 2, q2)
                gather_issue(i + 2, q2, p)
                pltpu.async_copy(e_slice(i + 2), eb[p], esem[p])
            pltpu.make_async_copy(mb[p], acc.at[dstb.at[q]], ssem[p]).wait()
            if pf4:
                idx_issue(i + 4, q)

        for q in range(4):
            idx_issue(q, q)
        idx_wait(0, 0)
        gather_issue(0, 0, 0)
        pltpu.async_copy(e_slice(0), eb[0], esem[0])
        idx_wait(1, 1)
        gather_issue(1, 1, 1)
        pltpu.async_copy(e_slice(1), eb[1], esem[1])

        assert NCH % 4 == 0

        def quad(j, carry):
            i0 = j * 4
            for k in range(4):
                step(i0 + k, k % 2, k, True, True)
            return carry

        lax.fori_loop(0, NCH // 4 - 1, quad, 0)
        i0 = NCH - 4
        step(i0, 0, 0, True, False)
        step(i0 + 1, 1, 1, True, False)
        step(i0 + 2, 0, 2, False, False)
        step(i0 + 3, 1, 3, False, False)

        plsc.subcore_barrier()
        pltpu.sync_copy(acc.at[pl.ds(s * RPT, RPT)],
                        out_hbm.at[c, pl.ds(s * RPT, RPT)])

    return body(x, src1, dst1, e, zeros)


def _final_body(x_ref, p_ref, w_ref, b_ref, o_ref):
    h = x_ref[...] + p_ref[0] + p_ref[1]
    o_ref[...] = (
        jnp.dot(h, w_ref[...], preferred_element_type=jnp.float32)
        + b_ref[...]
    )


def _final(x, parts, W, b, block_n):
    N, D = x.shape
    grid = N // block_n
    return pl.pallas_call(
        _final_body,
        grid=(grid,),
        in_specs=[
            pl.BlockSpec((block_n, D), lambda i: (i, 0)),
            pl.BlockSpec((2, block_n, D), lambda i: (0, i, 0)),
            pl.BlockSpec((D, D), lambda i: (0, 0)),
            pl.BlockSpec((1, D), lambda i: (0, 0)),
        ],
        out_specs=pl.BlockSpec((block_n, D), lambda i: (i, 0)),
        out_shape=jax.ShapeDtypeStruct((N, D), jnp.float32),
    )(x, parts, W, b.reshape(1, D))


def kernel(x, edge_index, edge_attr, y, W_edge, b_edge, W, b):
    N, D = x.shape
    E = edge_index.shape[1]
    NW, NCH, CB = 32, 160, 64
    ep = NW * NCH * CB
    npad = -(-N // 128) * 128
    src = edge_index[0]
    dst = edge_index[1]
    pad = ep - E
    pidx = jnp.arange(pad, dtype=jnp.int32)
    src1 = jnp.concatenate([src, pidx % N])
    dst1 = jnp.concatenate([dst, N + pidx % (npad - N)])
    e = _edge_linear(edge_attr.T, W_edge, b_edge, ep, block_e=2560, cb=CB)
    zeros = jnp.zeros((npad, D), jnp.float32)
    parts = _sc_aggregate(x, src1, dst1, e, zeros, NCH, CB)
    pred = _final(x, parts, W, b, block_n=1000)
    return (pred, y)


# speedup vs baseline: 2.4047x; 1.1086x over previous
import functools

import jax
import jax.numpy as jnp
from jax import lax
from jax.experimental import pallas as pl
from jax.experimental.pallas import tpu as pltpu
from jax.experimental.pallas import tpu_sc as plsc


def _pack_bf16_pairs(lo_f32, hi_f32):
    lo = lax.bitcast_convert_type(
        lo_f32.astype(jnp.bfloat16), jnp.uint16).astype(jnp.uint32)
    hi = lax.bitcast_convert_type(
        hi_f32.astype(jnp.bfloat16), jnp.uint16).astype(jnp.uint32)
    return ((hi << 16) | lo).astype(jnp.int32)


def _make_edge_lin_body(cb):
    def body(at_ref, wlo_ref, whi_ref, blo_ref, bhi_ref, o_ref):
        dn = (((0,), (0,)), ((), ()))
        e_lo = lax.dot_general(at_ref[...], wlo_ref[...], dn,
                               preferred_element_type=jnp.float32) + blo_ref[...]
        e_hi = lax.dot_general(at_ref[...], whi_ref[...], dn,
                               preferred_element_type=jnp.float32) + bhi_ref[...]
        packed = _pack_bf16_pairs(e_lo, e_hi)
        be, dh = packed.shape
        pr = packed.reshape(be // cb, 2, cb // 2, dh)
        o_ref[...] = jnp.concatenate([pr[:, 0], pr[:, 1]],
                                     axis=-1).reshape(be // 2, 2 * dh)
    return body


def _edge_linear(edge_attr_t, W_edge, b_edge, ep, block_e, cb, base_blk):
    DE, E = edge_attr_t.shape
    D = W_edge.shape[1]
    grid = ep // block_e
    return pl.pallas_call(
        _make_edge_lin_body(cb),
        grid=(grid,),
        in_specs=[
            pl.BlockSpec((DE, block_e), lambda i: (0, i + base_blk)),
            pl.BlockSpec((DE, D // 2), lambda i: (0, 0)),
            pl.BlockSpec((DE, D // 2), lambda i: (0, 0)),
            pl.BlockSpec((1, D // 2), lambda i: (0, 0)),
            pl.BlockSpec((1, D // 2), lambda i: (0, 0)),
        ],
        out_specs=pl.BlockSpec((block_e // 2, D), lambda i: (i, 0)),
        out_shape=jax.ShapeDtypeStruct((ep // 2, D), jnp.int32),
    )(edge_attr_t, W_edge[:, :D // 2], W_edge[:, D // 2:],
      b_edge[:D // 2].reshape(1, D // 2), b_edge[D // 2:].reshape(1, D // 2))


def _sc_aggregate(x, src1, dst1, e, zeros, nch, cb):
    N = x.shape[0]
    NPAD, D = zeros.shape
    NCH, CB = nch, cb
    info = plsc.get_sparse_core_info()
    NC, NS = info.num_cores, info.num_subcores
    NW = NC * NS
    EPT = NCH * CB
    assert src1.shape[0] == NW * EPT and NCH % 2 == 0 and CB % 8 == 0
    assert NPAD % (8 * NS) == 0
    RPT = NPAD // NS

    mesh = plsc.VectorSubcoreMesh(core_axis_name="c", subcore_axis_name="s")

    @functools.partial(
        pl.kernel,
        out_type=jax.ShapeDtypeStruct((NC, NPAD, D), jnp.float32),
        mesh=mesh,
        scratch_types=[
            pltpu.VMEM((4, CB), jnp.int32),
            pltpu.VMEM((4, CB), jnp.int32),
            pltpu.VMEM((CB, D), jnp.float32),
            pltpu.VMEM((CB, D), jnp.float32),
            pltpu.VMEM((CB // 2, D), jnp.int32),
            pltpu.VMEM((CB // 2, D), jnp.int32),
            pltpu.VMEM((CB, D), jnp.float32),
            pltpu.VMEM((CB, D), jnp.float32),
            pltpu.VMEM_SHARED((NPAD, D), jnp.float32),
            pltpu.SemaphoreType.DMA,
            pltpu.SemaphoreType.DMA,
            pltpu.SemaphoreType.DMA,
            pltpu.SemaphoreType.DMA,
            pltpu.SemaphoreType.DMA,
            pltpu.SemaphoreType.DMA,
            pltpu.SemaphoreType.DMA,
            pltpu.SemaphoreType.DMA,
            pltpu.SemaphoreType.DMA,
            pltpu.SemaphoreType.DMA,
        ],
    )
    def body(x_hbm, src_hbm, dst_hbm, e_hbm, zero_hbm, out_hbm,
             srcb, dstb, xb0, xb1, eb0, eb1, mb0, mb1, acc,
             gsem0, gsem1, esem0, esem1, ssem0, ssem1,
             isem0, isem1, isem2, isem3):
        c = lax.axis_index("c")
        s = lax.axis_index("s")
        wid = c * NS + s
        xb = (xb0, xb1)
        eb = (eb0, eb1)
        mb = (mb0, mb1)
        gsem = (gsem0, gsem1)
        esem = (esem0, esem1)
        ssem = (ssem0, ssem1)
        isem = (isem0, isem1, isem2, isem3)

        pltpu.sync_copy(zero_hbm.at[pl.ds(s * RPT, RPT)],
                        acc.at[pl.ds(s * RPT, RPT)])
        plsc.subcore_barrier()

        ebase = wid * EPT

        def i_slice(hbm, i):
            return hbm.at[pl.ds(pl.multiple_of(ebase + i * CB, 8), CB)]

        def e_slice(i):
            return e_hbm.at[
                pl.ds(pl.multiple_of((ebase + i * CB) // 2, 8), CB // 2)]

        def idx_issue(i, q):
            pltpu.async_copy(i_slice(src_hbm, i), srcb.at[q], isem[q])
            pltpu.async_copy(i_slice(dst_hbm, i), dstb.at[q], isem[q])

        def idx_wait(i, q):
            pltpu.make_async_copy(i_slice(src_hbm, i), srcb.at[q],
                                  isem[q]).wait()
            pltpu.make_async_copy(i_slice(dst_hbm, i), dstb.at[q],
                                  isem[q]).wait()

        def gather_issue(i, q, p):
            pltpu.async_copy(x_hbm.at[srcb.at[q]], xb[p], gsem[p])

        def compute(p):
            xbp = xb[p]
            ebp = eb[p]
            mbp = mb[p]

            mask = jnp.full((16,), -0x10000, jnp.int32)

            def as_f32(bits):
                return lax.bitcast_convert_type(bits, jnp.float32)

            def rowpair(k, carry):
                for je in range(2):
                    r = k + je * (CB // 2)
                    for j in range(D // 32):
                        lo_sl = pl.ds(j * 16, 16)
                        hi_sl = pl.ds(D // 2 + j * 16, 16)
                        ew = ebp[k, pl.ds(je * (D // 2) + j * 16, 16)]
                        lo = xbp[r, lo_sl] + as_f32(ew << 16)
                        hi = xbp[r, hi_sl] + as_f32(ew & mask)
                        mbp[r, lo_sl] = jnp.maximum(lo, 0.0)
                        mbp[r, hi_sl] = jnp.maximum(hi, 0.0)
                return carry

            lax.fori_loop(0, CB // 2, rowpair, 0)

        def step(i, p, q, pf2, pf4):
            pltpu.make_async_copy(x_hbm.at[srcb.at[q]], xb[p], gsem[p]).wait()
            pltpu.make_async_copy(e_slice(i), eb[p], esem[p]).wait()
            compute(p)
            pltpu.async_copy(mb[p], acc.at[dstb.at[q]], ssem[p], add=True)
            if pf2:
                q2 = (q + 2) % 4
                idx_wait(i + 2, q2)
                gather_issue(i + 2, q2, p)
                pltpu.async_copy(e_slice(i + 2), eb[p], esem[p])
            pltpu.make_async_copy(mb[p], acc.at[dstb.at[q]], ssem[p]).wait()
            if pf4:
                idx_issue(i + 4, q)

        for q in range(4):
            idx_issue(q, q)
        idx_wait(0, 0)
        gather_issue(0, 0, 0)
        pltpu.async_copy(e_slice(0), eb[0], esem[0])
        idx_wait(1, 1)
        gather_issue(1, 1, 1)
        pltpu.async_copy(e_slice(1), eb[1], esem[1])

        assert NCH % 4 == 0

        def quad(j, carry):
            i0 = j * 4
            for k in range(4):
                step(i0 + k, k % 2, k, True, True)
            return carry

        lax.fori_loop(0, NCH // 4 - 1, quad, 0)
        i0 = NCH - 4
        step(i0, 0, 0, True, False)
        step(i0 + 1, 1, 1, True, False)
        step(i0 + 2, 0, 2, False, False)
        step(i0 + 3, 1, 3, False, False)

        plsc.subcore_barrier()
        pltpu.sync_copy(acc.at[pl.ds(s * RPT, RPT)],
                        out_hbm.at[c, pl.ds(s * RPT, RPT)])

    return body(x, src1, dst1, e, zeros)


def _final_body(x_ref, p1_ref, p2_ref, w_ref, b_ref, o_ref):
    h = x_ref[...] + (p1_ref[0] + p1_ref[1]) + (p2_ref[0] + p2_ref[1])
    o_ref[...] = (
        jnp.dot(h, w_ref[...], preferred_element_type=jnp.float32)
        + b_ref[...]
    )


def _final(x, parts1, parts2, W, b, block_n):
    N, D = x.shape
    grid = N // block_n
    return pl.pallas_call(
        _final_body,
        grid=(grid,),
        in_specs=[
            pl.BlockSpec((block_n, D), lambda i: (i, 0)),
            pl.BlockSpec((2, block_n, D), lambda i: (0, i, 0)),
            pl.BlockSpec((2, block_n, D), lambda i: (0, i, 0)),
            pl.BlockSpec((D, D), lambda i: (0, 0)),
            pl.BlockSpec((1, D), lambda i: (0, 0)),
        ],
        out_specs=pl.BlockSpec((block_n, D), lambda i: (i, 0)),
        out_shape=jax.ShapeDtypeStruct((N, D), jnp.float32),
    )(x, parts1, parts2, W, b.reshape(1, D))


def kernel(x, edge_index, edge_attr, y, W_edge, b_edge, W, b):
    N, D = x.shape
    E = edge_index.shape[1]
    NW, NCH, CB = 32, 160, 64
    ep = NW * NCH * CB
    npad = -(-N // 128) * 128
    src = edge_index[0]
    dst = edge_index[1]
    pad = ep - E
    pidx = jnp.arange(pad, dtype=jnp.int32)
    src1 = jnp.concatenate([src, pidx % N])
    dst1 = jnp.concatenate([dst, N + pidx % (npad - N)])
    block_e = 2560
    eh = ep // 2
    zeros = jnp.zeros((npad, D), jnp.float32)
    at = edge_attr.T
    e1 = _edge_linear(at, W_edge, b_edge, eh, block_e, CB, base_blk=0)
    e2 = _edge_linear(at, W_edge, b_edge, eh, block_e, CB,
                      base_blk=eh // block_e)
    parts1 = _sc_aggregate(x, src1[:eh], dst1[:eh], e1, zeros, NCH // 2, CB)
    parts2 = _sc_aggregate(x, src1[eh:], dst1[eh:], e2, zeros, NCH // 2, CB)
    pred = _final(x, parts1, parts2, W, b, block_n=1000)
    return (pred, y)


# speedup vs baseline: 2.4795x; 1.0311x over previous
import functools

import jax
import jax.numpy as jnp
from jax import lax
from jax.experimental import pallas as pl
from jax.experimental.pallas import tpu as pltpu
from jax.experimental.pallas import tpu_sc as plsc


def _pack_bf16_pairs(lo_f32, hi_f32):
    lo = lax.bitcast_convert_type(lo_f32, jnp.uint32) >> 16
    hi = lax.bitcast_convert_type(hi_f32, jnp.uint32) & jnp.uint32(0xFFFF0000)
    return lax.bitcast_convert_type(hi | lo, jnp.int32)


def _make_edge_lin_body(cb):
    def body(at_ref, wlo_ref, whi_ref, blo_ref, bhi_ref, o_ref):
        dn = (((0,), (0,)), ((), ()))
        e_lo = (lax.dot_general(at_ref[...], wlo_ref[...], dn,
                                preferred_element_type=jnp.float32)
                + blo_ref[...])
        e_hi = (lax.dot_general(at_ref[...], whi_ref[...], dn,
                                preferred_element_type=jnp.float32)
                + bhi_ref[...])
        packed = _pack_bf16_pairs(e_lo, e_hi)
        be, dh = packed.shape
        pr = packed.reshape(be // cb, 2, cb // 2, dh)
        o_ref[...] = jnp.concatenate([pr[:, 0], pr[:, 1]],
                                     axis=-1).reshape(be // 2, 2 * dh)
    return body


def _edge_linear(edge_attr_t, W_edge, b_edge, ep, block_e, cb, base_blk):
    DE, E = edge_attr_t.shape
    D = W_edge.shape[1]
    grid = ep // block_e
    return pl.pallas_call(
        _make_edge_lin_body(cb),
        grid=(grid,),
        in_specs=[
            pl.BlockSpec((DE, block_e), lambda i: (0, i + base_blk)),
            pl.BlockSpec((DE, D // 2), lambda i: (0, 0)),
            pl.BlockSpec((DE, D // 2), lambda i: (0, 0)),
            pl.BlockSpec((1, D // 2), lambda i: (0, 0)),
            pl.BlockSpec((1, D // 2), lambda i: (0, 0)),
        ],
        out_specs=pl.BlockSpec((block_e // 2, D), lambda i: (i, 0)),
        out_shape=jax.ShapeDtypeStruct((ep // 2, D), jnp.int32),
    )(edge_attr_t, W_edge[:, :D // 2], W_edge[:, D // 2:],
      b_edge[:D // 2].reshape(1, D // 2), b_edge[D // 2:].reshape(1, D // 2))


def _sc_aggregate(x, src1, dst1, e, npad, nch, cb):
    N, D = x.shape
    NPAD = npad
    NCH, CB = nch, cb
    info = plsc.get_sparse_core_info()
    NC, NS = info.num_cores, info.num_subcores
    NW = NC * NS
    EPT = NCH * CB
    assert src1.shape[0] == NW * EPT and NCH % 2 == 0 and CB % 8 == 0
    assert NPAD % (8 * NS) == 0
    RPT = NPAD // NS

    mesh = plsc.VectorSubcoreMesh(core_axis_name="c", subcore_axis_name="s")

    @functools.partial(
        pl.kernel,
        out_type=jax.ShapeDtypeStruct((NC, NPAD, D), jnp.float32),
        mesh=mesh,
        scratch_types=[
            pltpu.VMEM((4, CB), jnp.int32),
            pltpu.VMEM((4, CB), jnp.int32),
            pltpu.VMEM((CB, D), jnp.float32),
            pltpu.VMEM((CB, D), jnp.float32),
            pltpu.VMEM((CB // 2, D), jnp.int32),
            pltpu.VMEM((CB // 2, D), jnp.int32),
            pltpu.VMEM((CB, D), jnp.float32),
            pltpu.VMEM((CB, D), jnp.float32),
            pltpu.VMEM_SHARED((NPAD, D), jnp.float32),
            pltpu.SemaphoreType.DMA,
            pltpu.SemaphoreType.DMA,
            pltpu.SemaphoreType.DMA,
            pltpu.SemaphoreType.DMA,
            pltpu.SemaphoreType.DMA,
            pltpu.SemaphoreType.DMA,
            pltpu.SemaphoreType.DMA,
            pltpu.SemaphoreType.DMA,
            pltpu.SemaphoreType.DMA,
            pltpu.SemaphoreType.DMA,
        ],
    )
    def body(x_hbm, src_hbm, dst_hbm, e_hbm, out_hbm,
             srcb, dstb, xb0, xb1, eb0, eb1, mb0, mb1, acc,
             gsem0, gsem1, esem0, esem1, ssem0, ssem1,
             isem0, isem1, isem2, isem3):
        c = lax.axis_index("c")
        s = lax.axis_index("s")
        wid = c * NS + s
        xb = (xb0, xb1)
        eb = (eb0, eb1)
        mb = (mb0, mb1)
        gsem = (gsem0, gsem1)
        esem = (esem0, esem1)
        ssem = (ssem0, ssem1)
        isem = (isem0, isem1, isem2, isem3)

        def zrow(r, carry):
            for j in range(D // 16):
                mb0[r, pl.ds(j * 16, 16)] = jnp.zeros((16,), jnp.float32)
            return carry

        lax.fori_loop(0, CB, zrow, 0)
        zbase = s * RPT
        for t in range(RPT // CB):
            pltpu.sync_copy(mb0, acc.at[pl.ds(zbase + t * CB, CB)])
        if RPT % CB:
            pltpu.sync_copy(mb0.at[pl.ds(0, RPT % CB)],
                            acc.at[pl.ds(zbase + (RPT // CB) * CB, RPT % CB)])
        plsc.subcore_barrier()

        ebase = wid * EPT

        def i_slice(hbm, i):
            return hbm.at[pl.ds(pl.multiple_of(ebase + i * CB, 8), CB)]

        def e_slice(i):
            return e_hbm.at[
                pl.ds(pl.multiple_of((ebase + i * CB) // 2, 8), CB // 2)]

        def idx_issue(i, q):
            pltpu.async_copy(i_slice(src_hbm, i), srcb.at[q], isem[q])
            pltpu.async_copy(i_slice(dst_hbm, i), dstb.at[q], isem[q])

        def idx_wait(i, q):
            pltpu.make_async_copy(i_slice(src_hbm, i), srcb.at[q],
                                  isem[q]).wait()
            pltpu.make_async_copy(i_slice(dst_hbm, i), dstb.at[q],
                                  isem[q]).wait()

        def gather_issue(i, q, p):
            pltpu.async_copy(x_hbm.at[srcb.at[q]], xb[p], gsem[p])

        def compute(p):
            xbp = xb[p]
            ebp = eb[p]
            mbp = mb[p]

            mask = jnp.full((16,), -0x10000, jnp.int32)

            def as_f32(bits):
                return lax.bitcast_convert_type(bits, jnp.float32)

            def rowpair(k, carry):
                for je in range(2):
                    r = k + je * (CB // 2)
                    for j in range(D // 32):
                        lo_sl = pl.ds(j * 16, 16)
                        hi_sl = pl.ds(D // 2 + j * 16, 16)
                        ew = ebp[k, pl.ds(je * (D // 2) + j * 16, 16)]
                        lo = xbp[r, lo_sl] + as_f32(ew << 16)
                        hi = xbp[r, hi_sl] + as_f32(ew & mask)
                        mbp[r, lo_sl] = jnp.maximum(lo, 0.0)
                        mbp[r, hi_sl] = jnp.maximum(hi, 0.0)
                return carry

            lax.fori_loop(0, CB // 2, rowpair, 0)

        def step(i, p, q, pf2, pf4):
            pltpu.make_async_copy(x_hbm.at[srcb.at[q]], xb[p], gsem[p]).wait()
            pltpu.make_async_copy(e_slice(i), eb[p], esem[p]).wait()
            compute(p)
            pltpu.async_copy(mb[p], acc.at[dstb.at[q]], ssem[p], add=True)
            if pf2:
                q2 = (q + 2) % 4
                idx_wait(i + 2, q2)
                gather_issue(i + 2, q2, p)
                pltpu.async_copy(e_slice(i + 2), eb[p], esem[p])
            pltpu.make_async_copy(mb[p], acc.at[dstb.at[q]], ssem[p]).wait()
            if pf4:
                idx_issue(i + 4, q)

        for q in range(4):
            idx_issue(q, q)
        idx_wait(0, 0)
        gather_issue(0, 0, 0)
        pltpu.async_copy(e_slice(0), eb[0], esem[0])
        idx_wait(1, 1)
        gather_issue(1, 1, 1)
        pltpu.async_copy(e_slice(1), eb[1], esem[1])

        assert NCH % 4 == 0

        def quad(j, carry):
            i0 = j * 4
            for k in range(4):
                step(i0 + k, k % 2, k, True, True)
            return carry

        lax.fori_loop(0, NCH // 4 - 1, quad, 0)
        i0 = NCH - 4
        step(i0, 0, 0, True, False)
        step(i0 + 1, 1, 1, True, False)
        step(i0 + 2, 0, 2, False, False)
        step(i0 + 3, 1, 3, False, False)

        plsc.subcore_barrier()
        pltpu.sync_copy(acc.at[pl.ds(s * RPT, RPT)],
                        out_hbm.at[c, pl.ds(s * RPT, RPT)])

    return body(x, src1, dst1, e)


def _final_body(x_ref, p1_ref, p2_ref, w_ref, b_ref, o_ref):
    h = x_ref[...] + (p1_ref[0] + p1_ref[1]) + (p2_ref[0] + p2_ref[1])
    o_ref[...] = (
        jnp.dot(h, w_ref[...], preferred_element_type=jnp.float32)
        + b_ref[...]
    )


def _final(x, parts1, parts2, W, b, block_n):
    N, D = x.shape
    grid = N // block_n
    return pl.pallas_call(
        _final_body,
        grid=(grid,),
        in_specs=[
            pl.BlockSpec((block_n, D), lambda i: (i, 0)),
            pl.BlockSpec((2, block_n, D), lambda i: (0, i, 0)),
            pl.BlockSpec((2, block_n, D), lambda i: (0, i, 0)),
            pl.BlockSpec((D, D), lambda i: (0, 0)),
            pl.BlockSpec((1, D), lambda i: (0, 0)),
        ],
        out_specs=pl.BlockSpec((block_n, D), lambda i: (i, 0)),
        out_shape=jax.ShapeDtypeStruct((N, D), jnp.float32),
    )(x, parts1, parts2, W, b.reshape(1, D))


def kernel(x, edge_index, edge_attr, y, W_edge, b_edge, W, b):
    N, D = x.shape
    E = edge_index.shape[1]
    NW, NCH, CB = 32, 160, 64
    ep = NW * NCH * CB
    npad = -(-N // 128) * 128
    src = edge_index[0]
    dst = edge_index[1]
    pad = ep - E
    pidx = jnp.arange(pad, dtype=jnp.int32)
    src1 = jnp.concatenate([src, pidx % N])
    dst1 = jnp.concatenate([dst, N + pidx % (npad - N)])
    block_e = 2560
    eh = ep // 2
    at = edge_attr.T
    e1 = _edge_linear(at, W_edge, b_edge, eh, block_e, CB, base_blk=0)
    e2 = _edge_linear(at, W_edge, b_edge, eh, block_e, CB,
                      base_blk=eh // block_e)
    parts1 = _sc_aggregate(x, src1[:eh], dst1[:eh], e1, npad, NCH // 2, CB)
    parts2 = _sc_aggregate(x, src1[eh:], dst1[eh:], e2, npad, NCH // 2, CB)
    pred = _final(x, parts1, parts2, W, b, block_n=1000)
    return (pred, y)
